# baseline (speedup 1.0000x reference)
import jax
import jax.numpy as jnp
from jax.experimental import pallas as pl

N = 10000
EPS = 1e-5


def _ident_kernel(x_ref, o_ref):
    o_ref[...] = x_ref[...]


def _edgeconv(x, src, dst, Wa, ba, Wb, bb):
    x_i = x[dst]
    x_j = x[src]
    h = jnp.concatenate([x_i, x_j - x_i], axis=-1)
    h = jax.nn.relu(h @ Wa.T + ba) @ Wb.T + bb
    agg = jax.ops.segment_max(h, dst, num_segments=N)
    return jnp.where(jnp.isfinite(agg), agg, 0.0)


def _bn(x, g, b):
    return (x / jnp.sqrt(1.0 + EPS)) * g + b


def kernel(x, edge_index, W1a, b1a, W1b, b1b, W2a, b2a, W2b, b2b, W3a, b3a, W3b, b3b, g1, be1, g2, be2, g3, be3, L1w, L1b, L2w, L2b, L3w, L3b, L4w, L4b):
    src = edge_index[0]
    dst = edge_index[1]
    h = _edgeconv(x, src, dst, W1a, b1a, W1b, b1b)
    h = jax.nn.relu(_bn(h, g1, be1))
    h = _edgeconv(h, src, dst, W2a, b2a, W2b, b2b)
    h = jax.nn.relu(_bn(h, g2, be2))
    h = _edgeconv(h, src, dst, W3a, b3a, W3b, b3b)
    h = jax.nn.relu(_bn(h, g3, be3))
    h = jax.nn.relu(h @ L1w.T + L1b)
    h = jax.nn.relu(h @ L2w.T + L2b)
    h = jax.nn.relu(h @ L3w.T + L3b)
    h = h @ L4w.T + L4b
    return pl.pallas_call(
        _ident_kernel,
        out_shape=jax.ShapeDtypeStruct(h.shape, h.dtype),
    )(h)


# speedup vs baseline: 2.7942x; 2.7942x over previous
import functools

import jax
import jax.numpy as jnp
from jax import lax
from jax.experimental import pallas as pl
from jax.experimental.pallas import tpu as pltpu
from jax.experimental.pallas import tpu_sc as plsc

N = 10000
E = 320000
IN = 128
HC = 64
TW = 2 * HC
EPS = 1e-5

NC, NS = 2, 16
NW = NC * NS
EPW = E // NW
GK = 400
SK = 3200
LANES = 16

_MESH = plsc.VectorSubcoreMesh(
    core_axis_name="c", subcore_axis_name="s", num_cores=NC, num_subcores=NS)

_BN_S = (1.0 + EPS) ** -0.5


def _tc_pre1_body(x_ref, wd_ref, wr_ref, ba_ref, t_out):
    xb = x_ref[...]
    bmat = lax.dot_general(xb, wr_ref[...], (((1,), (1,)), ((), ())),
                           preferred_element_type=jnp.float32)
    c = lax.dot_general(xb, wd_ref[...], (((1,), (1,)), ((), ())),
                        preferred_element_type=jnp.float32) \
        + ba_ref[...][None, :]
    t_out[...] = jnp.concatenate([c, bmat], axis=1)


def _tc_pre1(x, wd, wr, ba):
    return pl.pallas_call(
        _tc_pre1_body,
        out_shape=jax.ShapeDtypeStruct((N, TW), jnp.float32),
    )(x, wd, wr, ba)


def _tc_pre_body(agg_ref, g_ref, be_ref, wd_ref, wr_ref, ba_ref, t_out):
    a = jnp.max(agg_ref[...], axis=0)
    a = jnp.where(jnp.isfinite(a), a, 0.0)
    s = g_ref[...] * _BN_S
    h = jnp.maximum(a * s[:, None] + be_ref[...][:, None], 0.0)
    bmat = lax.dot_general(h, wr_ref[...], (((0,), (1,)), ((), ())),
                           preferred_element_type=jnp.float32)
    c = lax.dot_general(h, wd_ref[...], (((0,), (1,)), ((), ())),
                        preferred_element_type=jnp.float32) \
        + ba_ref[...][None, :]
    t_out[...] = jnp.concatenate([c, bmat], axis=1)


def _tc_pre(agg_t, g, be, wd, wr, ba):
    return pl.pallas_call(
        _tc_pre_body,
        out_shape=jax.ShapeDtypeStruct((N, TW), jnp.float32),
    )(agg_t, g, be, wd, wr, ba)


_EB = 6400


def _tc_edge_body(h_ref, w_ref, b_ref, o_ref):
    hb = h_ref[...]
    m = lax.dot_general(w_ref[...], hb, (((1,), (1,)), ((), ())),
                        preferred_element_type=jnp.float32)
    o_ref[...] = m + b_ref[...][:, None]


def _tc_edge(h, wb_pad, bb):
    grid = E // _EB
    return pl.pallas_call(
        _tc_edge_body,
        grid=(grid,),
        in_specs=[
            pl.BlockSpec((_EB, TW), lambda i: (i, 0)),
            pl.BlockSpec((HC, TW), lambda i: (0, 0)),
            pl.BlockSpec((HC,), lambda i: (0,)),
        ],
        out_specs=pl.BlockSpec((HC, _EB), lambda i: (0, i)),
        out_shape=jax.ShapeDtypeStruct((HC, E), jnp.float32),
    )(h, wb_pad, bb)


def _tc_head_body(agg_ref, g_ref, be_ref, w1_ref, b1_ref, w2_ref, b2_ref,
                  w3_ref, b3_ref, w4_ref, b4_ref, o_ref):
    a = jnp.max(agg_ref[...], axis=0)
    a = jnp.where(jnp.isfinite(a), a, 0.0)
    s = g_ref[...] * _BN_S
    h = jnp.maximum(a * s[:, None] + be_ref[...][:, None], 0.0)
    h = jnp.maximum(lax.dot_general(w1_ref[...], h, (((1,), (0,)), ((), ())),
                                    preferred_element_type=jnp.float32)
                    + b1_ref[...][:, None], 0.0)
    h = jnp.maximum(lax.dot_general(w2_ref[...], h, (((1,), (0,)), ((), ())),
                                    preferred_element_type=jnp.float32)
                    + b2_ref[...][:, None], 0.0)
    h = jnp.maximum(lax.dot_general(w3_ref[...], h, (((1,), (0,)), ((), ())),
                                    preferred_element_type=jnp.float32)
                    + b3_ref[...][:, None], 0.0)
    o_ref[...] = lax.dot_general(h, w4_ref[...], (((0,), (1,)), ((), ())),
                                 preferred_element_type=jnp.float32) \
        + b4_ref[...][None, :]


def _tc_head(agg_t, g, be, w1, b1, w2, b2, w3, b3, w4, b4):
    return pl.pallas_call(
        _tc_head_body,
        out_shape=jax.ShapeDtypeStruct((N, w4.shape[0]), jnp.float32),
    )(agg_t, g, be, w1, b1, w2, b2, w3, b3, w4, b4)


def _sc_gather_body(src_hbm, dst_hbm, t_hbm, out_hbm,
                    idxd, idxs, bufd, bufs, sem1, sem2):
    wid = lax.axis_index("s") * NC + lax.axis_index("c")
    base_w = wid * EPW

    @pl.loop(0, EPW // GK)
    def _chunk(j):
        base = pl.multiple_of(base_w + j * GK, 8)
        pltpu.sync_copy(dst_hbm.at[pl.ds(base, GK)], idxd)
        pltpu.sync_copy(src_hbm.at[pl.ds(base, GK)], idxs)
        cp1 = pltpu.async_copy(t_hbm.at[idxd], bufd, sem1)
        cp2 = pltpu.async_copy(t_hbm.at[idxs], bufs, sem2)
        cp1.wait()
        cp2.wait()

        @pl.loop(0, GK)
        def _row(r):
            for c in range(HC // LANES):
                sl = pl.ds(c * LANES, LANES)
                sr = pl.ds(HC + c * LANES, LANES)
                bufd[r, sl] = jnp.maximum(bufd[r, sl] + bufs[r, sr], 0.0)

        pltpu.sync_copy(bufd, out_hbm.at[pl.ds(base, GK), :])


def _sc_gather(src, dst, t_tab):
    f = functools.partial(
        pl.kernel,
        mesh=_MESH,
        out_type=jax.ShapeDtypeStruct((E, TW), jnp.float32),
        scratch_types=[
            pltpu.VMEM((GK,), jnp.int32),
            pltpu.VMEM((GK,), jnp.int32),
            pltpu.VMEM((GK, TW), jnp.float32),
            pltpu.VMEM((GK, TW), jnp.float32),
            pltpu.SemaphoreType.DMA,
            pltpu.SemaphoreType.DMA,
        ],
    )(_sc_gather_body)
    return f(src, dst, t_tab)


NFP = 8
NEP = 4
EPQ = E // NEP


def _sc_scatmax_body(dst_hbm, mt_hbm, out_hbm, idxb, vals, scr, *accs):
    wid = lax.axis_index("s") * NC + lax.axis_index("c")
    p = wid // NFP
    f0 = pl.multiple_of((wid % NFP) * NFP, 8)
    base_e = p * EPQ
    neg = jnp.full((LANES,), -jnp.inf, jnp.float32)

    @pl.loop(0, N // LANES)
    def _init(i):
        sl = pl.ds(i * LANES, LANES)
        for acc in accs:
            acc[sl] = neg

    @pl.loop(0, EPQ // SK)
    def _chunk(j):
        e0 = pl.multiple_of(base_e + j * SK, 128)
        pltpu.sync_copy(dst_hbm.at[pl.ds(e0, SK)], idxb)
        pltpu.sync_copy(mt_hbm.at[pl.ds(f0, NFP), pl.ds(e0, SK)], vals)

        @pl.loop(0, SK // LANES)
        def _grp(g):
            sl = pl.ds(g * LANES, LANES)
            idxv = idxb[sl]
            lanes = lax.iota(jnp.int32, LANES).astype(jnp.float32)
            plsc.store_scatter(scr, [idxv], lanes)
            rd = plsc.load_gather(scr, [idxv])
            ndup = jnp.sum((rd != lanes).astype(jnp.int32))

            @pl.when(ndup == 0)
            def _fast():
                for f, acc in enumerate(accs):
                    v = vals[f, sl]
                    cur = plsc.load_gather(acc, [idxv])
                    plsc.store_scatter(acc, [idxv], jnp.maximum(v, cur))

            @pl.when(ndup > 0)
            def _slow():
                for f, acc in enumerate(accs):
                    v = vals[f, sl]
                    cur = plsc.load_gather(acc, [idxv])
                    m = jnp.maximum(v, cur)
                    plsc.store_scatter(acc, [idxv], m)
                    back = plsc.load_gather(acc, [idxv])
                    cnt = jnp.sum((back < m).astype(jnp.int32))

                    def _cond(c):
                        return c > 0

                    def _body(c):
                        b1 = plsc.load_gather(acc, [idxv])
                        msk = b1 < m
                        plsc.store_scatter(acc, [idxv], m, mask=msk)
                        b2 = plsc.load_gather(acc, [idxv])
                        return jnp.sum((b2 < m).astype(jnp.int32))

                    lax.while_loop(_cond, _body, cnt)

    for f, acc in enumerate(accs):
        base = pl.multiple_of((p * HC + f0 + f) * N, 8)
        pltpu.sync_copy(acc, out_hbm.at[pl.ds(base, N)])


def _sc_scatmax(dst, m_t):
    f = functools.partial(
        pl.kernel,
        mesh=_MESH,
        compiler_params=pltpu.CompilerParams(needs_layout_passes=False),
        out_type=jax.ShapeDtypeStruct((NEP * HC * N,), jnp.float32),
        scratch_types=[
            pltpu.VMEM((SK,), jnp.int32),
            pltpu.VMEM((NFP, SK), jnp.float32),
            pltpu.VMEM((N,), jnp.float32),
        ] + [pltpu.VMEM((N,), jnp.float32)] * NFP,
    )(_sc_scatmax_body)
    return jnp.reshape(f(dst, m_t), (NEP, HC, N))


def _pad_w(wb):
    return jnp.concatenate([wb, jnp.zeros_like(wb)], axis=1)


def kernel(x, edge_index, W1a, b1a, W1b, b1b, W2a, b2a, W2b, b2b, W3a, b3a,
           W3b, b3b, g1, be1, g2, be2, g3, be3, L1w, L1b, L2w, L2b, L3w, L3b,
           L4w, L4b):
    src = edge_index[0]
    dst = edge_index[1]

    t_tab = _tc_pre1(x, W1a[:, :IN] - W1a[:, IN:], W1a[:, IN:], b1a)
    h_e = _sc_gather(src, dst, t_tab)
    m_t = _tc_edge(h_e, _pad_w(W1b), b1b)
    agg = _sc_scatmax(dst, m_t)

    t_tab = _tc_pre(agg, g1, be1, W2a[:, :HC] - W2a[:, HC:], W2a[:, HC:], b2a)
    h_e = _sc_gather(src, dst, t_tab)
    m_t = _tc_edge(h_e, _pad_w(W2b), b2b)
    agg = _sc_scatmax(dst, m_t)

    t_tab = _tc_pre(agg, g2, be2, W3a[:, :HC] - W3a[:, HC:], W3a[:, HC:], b3a)
    h_e = _sc_gather(src, dst, t_tab)
    m_t = _tc_edge(h_e, _pad_w(W3b), b3b)
    agg = _sc_scatmax(dst, m_t)

    return _tc_head(agg, g3, be3, L1w, L1b, L2w, L2b, L3w, L3b, L4w, L4b)
